# baseline (speedup 1.0000x reference)
import functools

import jax
import jax.numpy as jnp
from jax import lax
from jax.experimental import pallas as pl
from jax.experimental.pallas import tpu as pltpu
from jax.experimental.pallas import tpu_sc as plsc

_B, _C, _S = 4, 2048, 3
_V, _H = 1000, 128
_VP = 1024
_NC, _NS = 2, 16
_NW = _NC * _NS
_NCL = _B * _C
_CPW = _NCL // _NW
_DUMMY = _V

_SC_MESH = plsc.VectorSubcoreMesh(core_axis_name="c", subcore_axis_name="s")


def _sc_gather_body(table_hbm, fidx_hbm, ramps_hbm, gout_hbm,
                    idx_v, ridx_v, b0, b1, b2, shared_g, sem):
    cid = lax.axis_index("c")
    sid = lax.axis_index("s")
    chunk = cid * _NS + sid
    pltpu.sync_copy(fidx_hbm.at[pl.ds(chunk * 8, 8)], idx_v)
    pltpu.sync_copy(ramps_hbm.at[pl.ds(sid * 8, 8)], ridx_v)
    for hh in range(2):
        c0 = pltpu.async_copy(table_hbm.at[idx_v.at[0 + hh]], b0, sem)
        c1 = pltpu.async_copy(table_hbm.at[idx_v.at[2 + hh]], b1, sem)
        c2 = pltpu.async_copy(table_hbm.at[idx_v.at[4 + hh]], b2, sem)
        c0.wait()
        c1.wait()
        c2.wait()
        pltpu.sync_copy(b0, shared_g.at[pl.ds(sid * _CPW + hh * 128, 128)])
        pltpu.sync_copy(b1, shared_g.at[ridx_v.at[hh]], add=True)
        pltpu.sync_copy(b2, shared_g.at[ridx_v.at[hh]], add=True)
    pltpu.sync_copy(shared_g.at[pl.ds(sid * _CPW, _CPW)],
                    gout_hbm.at[pl.ds(chunk * _CPW, _CPW)])


_sc_gather = functools.partial(
    pl.kernel,
    mesh=_SC_MESH,
    out_type=jax.ShapeDtypeStruct((_NCL, _H), jnp.float32),
    scratch_types=[
        pltpu.VMEM((8, 128), jnp.int32),
        pltpu.VMEM((8, 128), jnp.int32),
        pltpu.VMEM((128, _H), jnp.float32),
        pltpu.VMEM((128, _H), jnp.float32),
        pltpu.VMEM((128, _H), jnp.float32),
        pltpu.VMEM_SHARED((_NS * _CPW, _H), jnp.float32),
        pltpu.SemaphoreType.DMA,
    ],
)(_sc_gather_body)


def _dedup_dests(iv, dv):
    for hh in range(2):
        for k in range(8):
            sl = pl.ds(k * 16, 16)
            i0 = iv[0 + hh, sl]
            i1 = iv[2 + hh, sl]
            i2 = iv[4 + hh, sl]
            dv[hh, sl] = jnp.where(i1 == i0, _DUMMY, i1)
            dv[2 + hh, sl] = jnp.where((i2 == i0) | (i2 == i1), _DUMMY, i2)


def _sc_scatter_c_body(t_hbm, sidx_hbm, z2d_hbm, e0_hbm, msum_hbm, cnt_hbm,
                       iv, dv, tb, e0_v, shared_m, shared_c, sem):
    cid = lax.axis_index("c")
    sid = lax.axis_index("s")
    chunk = cid * _NS + sid
    pltpu.sync_copy(z2d_hbm, shared_m.at[pl.ds(sid * 64, 64)])
    pltpu.sync_copy(z2d_hbm, shared_c.at[pl.ds(sid * 64, 64)])
    pltpu.sync_copy(t_hbm.at[pl.ds(chunk * _CPW, _CPW)], tb)
    pltpu.sync_copy(sidx_hbm.at[pl.ds(chunk * 8, 8)], iv)
    pltpu.sync_copy(e0_hbm, e0_v)
    _dedup_dests(iv, dv)
    plsc.subcore_barrier()
    for hh in range(2):
        src = tb.at[pl.ds(hh * 128, 128)]
        pltpu.sync_copy(src, shared_m.at[iv.at[hh]], add=True)
        pltpu.sync_copy(src, shared_m.at[dv.at[hh]], add=True)
        pltpu.sync_copy(src, shared_m.at[dv.at[2 + hh]], add=True)
        pltpu.sync_copy(e0_v, shared_c.at[iv.at[hh]], add=True)
        pltpu.sync_copy(e0_v, shared_c.at[dv.at[hh]], add=True)
        pltpu.sync_copy(e0_v, shared_c.at[dv.at[2 + hh]], add=True)
    plsc.subcore_barrier()
    pltpu.sync_copy(shared_m.at[pl.ds(sid * 64, 64)],
                    msum_hbm.at[pl.ds(cid * _VP + sid * 64, 64)])
    pltpu.sync_copy(shared_c.at[pl.ds(sid * 64, 64)],
                    cnt_hbm.at[pl.ds(cid * _VP + sid * 64, 64)])


_sc_scatter_c = functools.partial(
    pl.kernel,
    mesh=_SC_MESH,
    out_type=[
        jax.ShapeDtypeStruct((_NC * _VP, _H), jnp.float32),
        jax.ShapeDtypeStruct((_NC * _VP, _H), jnp.float32),
    ],
    scratch_types=[
        pltpu.VMEM((8, 128), jnp.int32),
        pltpu.VMEM((4, 128), jnp.int32),
        pltpu.VMEM((_CPW, _H), jnp.float32),
        pltpu.VMEM((128, _H), jnp.float32),
        pltpu.VMEM_SHARED((_VP, _H), jnp.float32),
        pltpu.VMEM_SHARED((_VP, _H), jnp.float32),
        pltpu.SemaphoreType.DMA,
    ],
)(_sc_scatter_c_body)


def _sc_scatter_nc_body(t_hbm, sidx_hbm, z2d_hbm, msum_hbm,
                        iv, dv, tb, shared_m, sem):
    cid = lax.axis_index("c")
    sid = lax.axis_index("s")
    chunk = cid * _NS + sid
    pltpu.sync_copy(z2d_hbm, shared_m.at[pl.ds(sid * 64, 64)])
    pltpu.sync_copy(t_hbm.at[pl.ds(chunk * _CPW, _CPW)], tb)
    pltpu.sync_copy(sidx_hbm.at[pl.ds(chunk * 8, 8)], iv)
    _dedup_dests(iv, dv)
    plsc.subcore_barrier()
    for hh in range(2):
        src = tb.at[pl.ds(hh * 128, 128)]
        pltpu.sync_copy(src, shared_m.at[iv.at[hh]], add=True)
        pltpu.sync_copy(src, shared_m.at[dv.at[hh]], add=True)
        pltpu.sync_copy(src, shared_m.at[dv.at[2 + hh]], add=True)
    plsc.subcore_barrier()
    pltpu.sync_copy(shared_m.at[pl.ds(sid * 64, 64)],
                    msum_hbm.at[pl.ds(cid * _VP + sid * 64, 64)])


_sc_scatter_nc = functools.partial(
    pl.kernel,
    mesh=_SC_MESH,
    out_type=jax.ShapeDtypeStruct((_NC * _VP, _H), jnp.float32),
    scratch_types=[
        pltpu.VMEM((8, 128), jnp.int32),
        pltpu.VMEM((4, 128), jnp.int32),
        pltpu.VMEM((_CPW, _H), jnp.float32),
        pltpu.VMEM_SHARED((_VP, _H), jnp.float32),
        pltpu.SemaphoreType.DMA,
    ],
)(_sc_scatter_nc_body)


_MMBLK = 1024


def _mm1_body(g_ref, wvc_ref, bvc_ref, wce_ref, bce_ref, wcv_ref, bcv_ref,
              t_ref):
    g = g_ref[...]
    h = jnp.dot(g, wvc_ref[...], preferred_element_type=jnp.float32) + bvc_ref[...]
    cl = jnp.dot(h, wce_ref[...], preferred_element_type=jnp.float32) + bce_ref[...]
    t_ref[...] = jnp.dot(cl, wcv_ref[...], preferred_element_type=jnp.float32) + bcv_ref[...]


def _mm2_body(g_ref, d_ref, wvc_ref, bvc_ref, wce_ref, bce_ref, wcv_ref,
              bcv_ref, t_ref):
    g = g_ref[...] + d_ref[...]
    h = jnp.dot(g, wvc_ref[...], preferred_element_type=jnp.float32) + bvc_ref[...]
    cl = jnp.dot(h, wce_ref[...], preferred_element_type=jnp.float32) + bce_ref[...]
    t_ref[...] = jnp.dot(cl, wcv_ref[...], preferred_element_type=jnp.float32) + bcv_ref[...]


def _mm_call(g, d2, wvc3, bvc2, wce_t, bce2, wcv_t, bcv2):
    nblk = _NCL // _MMBLK
    gspec = pl.BlockSpec((_MMBLK, _H), lambda i: (i, 0))
    wspec = pl.BlockSpec((_H, _H), lambda i: (0, 0))
    bspec = pl.BlockSpec((1, _H), lambda i: (0, 0))
    if d2 is None:
        body, ins = _mm1_body, (g,)
        specs = [gspec]
    else:
        body, ins = _mm2_body, (g, d2)
        specs = [gspec, gspec]
    return pl.pallas_call(
        body,
        grid=(nblk,),
        in_specs=specs + [wspec, bspec, wspec, bspec, wspec, bspec],
        out_specs=gspec,
        out_shape=jax.ShapeDtypeStruct((_NCL, _H), jnp.float32),
    )(*ins, wvc3, bvc2, wce_t, bce2, wcv_t, bcv2)


def _msg_body(msum_ref, cntp_ref, out_ref):
    cs = cntp_ref[0][:, 0:1] + cntp_ref[1][:, 0:1]
    scale = (cs > 0).astype(jnp.float32) / jnp.maximum(cs, 1.0)
    out_ref[...] = (msum_ref[0] + msum_ref[1]) * scale


def _msg_call(msum, cntp):
    return pl.pallas_call(
        _msg_body,
        out_shape=jax.ShapeDtypeStruct((_VP, _H), jnp.float32),
    )(msum, cntp)


def _head_body(vs_ref, m1_ref, m2_ref, cntp_ref, w1_ref, b1_ref, w2_ref,
               b2_ref, out_ref):
    cs = cntp_ref[0][:, 0:1] + cntp_ref[1][:, 0:1]
    scale = (cs > 0).astype(jnp.float32) / jnp.maximum(cs, 1.0)
    m = m1_ref[0] + m1_ref[1] + m2_ref[0] + m2_ref[1]
    vsn = vs_ref[0] + m * scale
    hh = jnp.maximum(
        jnp.dot(vsn, w1_ref[...], preferred_element_type=jnp.float32) + b1_ref[...],
        0.0)
    logit = jnp.sum(hh * w2_ref[...], axis=1, keepdims=True) + b2_ref[...]
    out_ref[0] = jax.nn.sigmoid(logit)


def _head_call(vs_p, m1, m2, cntp, w1_t, b12, w22, b22):
    mspec = pl.BlockSpec((2, _VP, _H), lambda b: (0, 0, 0))
    return pl.pallas_call(
        _head_body,
        grid=(_B,),
        in_specs=[
            pl.BlockSpec((1, _VP, _H), lambda b: (b, 0, 0)),
            mspec, mspec, mspec,
            pl.BlockSpec((_H, _H), lambda b: (0, 0)),
            pl.BlockSpec((1, _H), lambda b: (0, 0)),
            pl.BlockSpec((1, _H), lambda b: (0, 0)),
            pl.BlockSpec((1, 1), lambda b: (0, 0)),
        ],
        out_specs=pl.BlockSpec((1, _VP, 1), lambda b: (b, 0, 0)),
        out_shape=jax.ShapeDtypeStruct((_B, _VP, 1), jnp.float32),
    )(vs_p, m1, m2, cntp, w1_t, b12, w22, b22)


def _slot_major(ix):
    a = ix.reshape(_NW, 2, 128, _S).transpose(0, 3, 1, 2).reshape(_NW, 6, 128)
    a = jnp.pad(a, ((0, 0), (0, 2), (0, 0)))
    return a.reshape(_NW * 8, 128)


def kernel(clause_indices, variable_states, Wvc, bvc, Wce, bce, Wcv, bcv,
           W1, b1, W2, b2):
    idx = clause_indices
    sidx = _slot_major(idx)
    boff = (jnp.arange(_B, dtype=jnp.int32) * _V)[:, None, None]
    fidx = _slot_major(idx + boff)
    ramps = (jnp.arange(_NS * 2, dtype=jnp.int32)[:, None] * 128
             + jnp.arange(128, dtype=jnp.int32)[None, :])
    ramps = jnp.pad(ramps.reshape(_NS, 2, 128),
                    ((0, 0), (0, 6), (0, 0))).reshape(_NS * 8, 128)
    z2d = jnp.zeros((64, _H), jnp.float32)
    e0 = jnp.zeros((128, _H), jnp.float32).at[:, 0].set(1.0)
    vsf = variable_states.reshape(_B * _V, _H)
    vs_p = jnp.pad(variable_states, ((0, 0), (0, _VP - _V), (0, 0)))

    wvc3 = Wvc.T * (1.0 / _S)
    wce_t, wcv_t, w1_t = Wce.T, Wcv.T, W1.T
    bvc2, bce2, bcv2, b12 = (x.reshape(1, _H) for x in (bvc, bce, bcv, b1))
    w22 = W2.reshape(1, _H)
    b22 = b2.reshape(1, 1)

    g1 = _sc_gather(vsf, fidx, ramps)
    t1 = _mm_call(g1, None, wvc3, bvc2, wce_t, bce2, wcv_t, bcv2)
    msum1, cntp = _sc_scatter_c(t1, sidx, z2d, e0)
    msum1 = msum1.reshape(_NC, _VP, _H)
    cntp = cntp.reshape(_NC, _VP, _H)
    msg1 = _msg_call(msum1, cntp)

    d2 = _sc_gather(msg1, sidx, ramps)
    t2 = _mm_call(g1, d2, wvc3, bvc2, wce_t, bce2, wcv_t, bcv2)
    msum2 = _sc_scatter_nc(t2, sidx, z2d).reshape(_NC, _VP, _H)

    probs = _head_call(vs_p, msum1, msum2, cntp, w1_t, b12, w22, b22)
    return probs[:, :_V, 0]


# speedup vs baseline: 1.3351x; 1.3351x over previous
import functools

import jax
import jax.numpy as jnp
from jax import lax
from jax.experimental import pallas as pl
from jax.experimental.pallas import tpu as pltpu
from jax.experimental.pallas import tpu_sc as plsc

_B, _C, _S = 4, 2048, 3
_V, _H = 1000, 128
_VP = 1024
_NC, _NS = 2, 16
_NW = _NC * _NS
_NCL = _B * _C
_CPW = _NCL // _NW
_DUMMY = _V

_SC_MESH = plsc.VectorSubcoreMesh(core_axis_name="c", subcore_axis_name="s")


def _dedup_dests(iv, dv):
    for hh in range(2):
        for k in range(8):
            sl = pl.ds(k * 16, 16)
            i0 = iv[0 + hh, sl]
            i1 = iv[2 + hh, sl]
            i2 = iv[4 + hh, sl]
            dv[hh, sl] = jnp.where(i1 == i0, _DUMMY, i1)
            dv[2 + hh, sl] = jnp.where((i2 == i0) | (i2 == i1), _DUMMY, i2)


def _slot_sum(b0, b1, b2, lo):
    def body(r, carry):
        for k in range(8):
            sl = pl.ds(k * 16, 16)
            b0[r, sl] = b0[r, sl] + b1[r, sl] + b2[r, sl]
        return carry
    lax.fori_loop(lo, lo + 128, body, 0)


def _sc_fused_c_body(u_hbm, fidx_hbm, sidx_hbm, z2d_hbm, e0_hbm,
                     msum_hbm, cnt_hbm,
                     idx_v, siv, dv, b0, b1, b2, b3, e0_v,
                     shared_m, shared_c, semi, semg0, semg1):
    cid = lax.axis_index("c")
    sid = lax.axis_index("s")
    chunk = cid * _NS + sid
    pltpu.sync_copy(fidx_hbm.at[pl.ds(chunk * 8, 8)], idx_v)
    pltpu.sync_copy(sidx_hbm.at[pl.ds(chunk * 8, 8)], siv)
    zi0 = pltpu.async_copy(z2d_hbm, shared_m.at[pl.ds(sid * 64, 64)], semi)
    zi1 = pltpu.async_copy(z2d_hbm, shared_c.at[pl.ds(sid * 64, 64)], semi)
    zi2 = pltpu.async_copy(e0_hbm, e0_v, semi)
    g0 = [pltpu.async_copy(u_hbm.at[idx_v.at[s * 2]], b, semg0)
          for s, b in ((0, b0), (1, b1), (2, b2))]
    _dedup_dests(siv, dv)
    for g in g0:
        g.wait()
    _slot_sum(b0, b1, b2, 0)
    g1 = [pltpu.async_copy(u_hbm.at[idx_v.at[s * 2 + 1]], b, semg1)
          for s, b in ((0, b3), (1, b1), (2, b2))]
    zi0.wait()
    zi1.wait()
    zi2.wait()
    plsc.subcore_barrier()
    adds = [
        pltpu.async_copy(b0, shared_m.at[siv.at[0]], semi, add=True),
        pltpu.async_copy(b0, shared_m.at[dv.at[0]], semi, add=True),
        pltpu.async_copy(b0, shared_m.at[dv.at[2]], semi, add=True),
        pltpu.async_copy(e0_v, shared_c.at[siv.at[0]], semi, add=True),
        pltpu.async_copy(e0_v, shared_c.at[dv.at[0]], semi, add=True),
        pltpu.async_copy(e0_v, shared_c.at[dv.at[2]], semi, add=True),
    ]
    for g in g1:
        g.wait()
    _slot_sum(b3, b1, b2, 0)
    adds += [
        pltpu.async_copy(b3, shared_m.at[siv.at[1]], semi, add=True),
        pltpu.async_copy(b3, shared_m.at[dv.at[1]], semi, add=True),
        pltpu.async_copy(b3, shared_m.at[dv.at[3]], semi, add=True),
        pltpu.async_copy(e0_v, shared_c.at[siv.at[1]], semi, add=True),
        pltpu.async_copy(e0_v, shared_c.at[dv.at[1]], semi, add=True),
        pltpu.async_copy(e0_v, shared_c.at[dv.at[3]], semi, add=True),
    ]
    for a in adds:
        a.wait()
    plsc.subcore_barrier()
    pltpu.sync_copy(shared_m.at[pl.ds(sid * 64, 64)],
                    msum_hbm.at[pl.ds(cid * _VP + sid * 64, 64)])
    pltpu.sync_copy(shared_c.at[pl.ds(sid * 64, 64)],
                    cnt_hbm.at[pl.ds(cid * _VP + sid * 64, 64)])


_sc_fused_c = functools.partial(
    pl.kernel,
    mesh=_SC_MESH,
    out_type=[
        jax.ShapeDtypeStruct((_NC * _VP, _H), jnp.float32),
        jax.ShapeDtypeStruct((_NC * _VP, _H), jnp.float32),
    ],
    scratch_types=[
        pltpu.VMEM((8, 128), jnp.int32),
        pltpu.VMEM((8, 128), jnp.int32),
        pltpu.VMEM((4, 128), jnp.int32),
        pltpu.VMEM((128, _H), jnp.float32),
        pltpu.VMEM((128, _H), jnp.float32),
        pltpu.VMEM((128, _H), jnp.float32),
        pltpu.VMEM((128, _H), jnp.float32),
        pltpu.VMEM((128, _H), jnp.float32),
        pltpu.VMEM_SHARED((_VP, _H), jnp.float32),
        pltpu.VMEM_SHARED((_VP, _H), jnp.float32),
        pltpu.SemaphoreType.DMA,
        pltpu.SemaphoreType.DMA,
        pltpu.SemaphoreType.DMA,
    ],
)(_sc_fused_c_body)


def _sc_fused_nc_body(u_hbm, fidx_hbm, sidx_hbm, z2d_hbm, msum_hbm,
                      idx_v, siv, dv, b0, b1, b2, b3,
                      shared_m, semi, semg0, semg1):
    cid = lax.axis_index("c")
    sid = lax.axis_index("s")
    chunk = cid * _NS + sid
    pltpu.sync_copy(fidx_hbm.at[pl.ds(chunk * 8, 8)], idx_v)
    pltpu.sync_copy(sidx_hbm.at[pl.ds(chunk * 8, 8)], siv)
    zi0 = pltpu.async_copy(z2d_hbm, shared_m.at[pl.ds(sid * 64, 64)], semi)
    g0 = [pltpu.async_copy(u_hbm.at[idx_v.at[s * 2]], b, semg0)
          for s, b in ((0, b0), (1, b1), (2, b2))]
    _dedup_dests(siv, dv)
    for g in g0:
        g.wait()
    _slot_sum(b0, b1, b2, 0)
    g1 = [pltpu.async_copy(u_hbm.at[idx_v.at[s * 2 + 1]], b, semg1)
          for s, b in ((0, b3), (1, b1), (2, b2))]
    zi0.wait()
    plsc.subcore_barrier()
    adds = [
        pltpu.async_copy(b0, shared_m.at[siv.at[0]], semi, add=True),
        pltpu.async_copy(b0, shared_m.at[dv.at[0]], semi, add=True),
        pltpu.async_copy(b0, shared_m.at[dv.at[2]], semi, add=True),
    ]
    for g in g1:
        g.wait()
    _slot_sum(b3, b1, b2, 0)
    adds += [
        pltpu.async_copy(b3, shared_m.at[siv.at[1]], semi, add=True),
        pltpu.async_copy(b3, shared_m.at[dv.at[1]], semi, add=True),
        pltpu.async_copy(b3, shared_m.at[dv.at[3]], semi, add=True),
    ]
    for a in adds:
        a.wait()
    plsc.subcore_barrier()
    pltpu.sync_copy(shared_m.at[pl.ds(sid * 64, 64)],
                    msum_hbm.at[pl.ds(cid * _VP + sid * 64, 64)])


_sc_fused_nc = functools.partial(
    pl.kernel,
    mesh=_SC_MESH,
    out_type=jax.ShapeDtypeStruct((_NC * _VP, _H), jnp.float32),
    scratch_types=[
        pltpu.VMEM((8, 128), jnp.int32),
        pltpu.VMEM((8, 128), jnp.int32),
        pltpu.VMEM((4, 128), jnp.int32),
        pltpu.VMEM((128, _H), jnp.float32),
        pltpu.VMEM((128, _H), jnp.float32),
        pltpu.VMEM((128, _H), jnp.float32),
        pltpu.VMEM((128, _H), jnp.float32),
        pltpu.VMEM_SHARED((_VP, _H), jnp.float32),
        pltpu.SemaphoreType.DMA,
        pltpu.SemaphoreType.DMA,
        pltpu.SemaphoreType.DMA,
    ],
)(_sc_fused_nc_body)


def _u0_body(vs_ref, a_ref, out_ref):
    out_ref[0] = jnp.dot(vs_ref[0], a_ref[...],
                         preferred_element_type=jnp.float32)


def _u0_call(vs_p, a_m):
    return pl.pallas_call(
        _u0_body,
        grid=(_B,),
        in_specs=[
            pl.BlockSpec((1, _VP, _H), lambda b: (b, 0, 0)),
            pl.BlockSpec((_H, _H), lambda b: (0, 0)),
        ],
        out_specs=pl.BlockSpec((1, _VP, _H), lambda b: (b, 0, 0)),
        out_shape=jax.ShapeDtypeStruct((_B, _VP, _H), jnp.float32),
    )(vs_p, a_m)


def _msgu_body(u0_ref, msum_ref, cntp_ref, a_ref, d_ref, out_ref):
    cs = cntp_ref[0][:, 0:1] + cntp_ref[1][:, 0:1]
    has = (cs > 0).astype(jnp.float32)
    scale = has / jnp.maximum(cs, 1.0)
    msg = (msum_ref[0] + msum_ref[1]) * scale + has * d_ref[...]
    msga = jnp.dot(msg, a_ref[...], preferred_element_type=jnp.float32)
    out_ref[0] = u0_ref[0] + msga


def _msgu_call(u0, msum, cntp, a_m, d_r):
    mspec = pl.BlockSpec((2, _VP, _H), lambda b: (0, 0, 0))
    return pl.pallas_call(
        _msgu_body,
        grid=(_B,),
        in_specs=[
            pl.BlockSpec((1, _VP, _H), lambda b: (b, 0, 0)),
            mspec, mspec,
            pl.BlockSpec((_H, _H), lambda b: (0, 0)),
            pl.BlockSpec((1, _H), lambda b: (0, 0)),
        ],
        out_specs=pl.BlockSpec((1, _VP, _H), lambda b: (b, 0, 0)),
        out_shape=jax.ShapeDtypeStruct((_B, _VP, _H), jnp.float32),
    )(u0, msum, cntp, a_m, d_r)


def _head_body(vs_ref, m1_ref, m2_ref, cntp_ref, d_ref, w1_ref, b1_ref,
               w2_ref, b2_ref, out_ref):
    cs = cntp_ref[0][:, 0:1] + cntp_ref[1][:, 0:1]
    has = (cs > 0).astype(jnp.float32)
    scale = has / jnp.maximum(cs, 1.0)
    m = m1_ref[0] + m1_ref[1] + m2_ref[0] + m2_ref[1]
    vsn = vs_ref[0] + m * scale + has * (2.0 * d_ref[...])
    hh = jnp.maximum(
        jnp.dot(vsn, w1_ref[...], preferred_element_type=jnp.float32) + b1_ref[...],
        0.0)
    logit = jnp.sum(hh * w2_ref[...], axis=1, keepdims=True) + b2_ref[...]
    out_ref[0] = jax.nn.sigmoid(logit)


def _head_call(vs_p, m1, m2, cntp, d_r, w1_t, b12, w22, b22):
    mspec = pl.BlockSpec((2, _VP, _H), lambda b: (0, 0, 0))
    return pl.pallas_call(
        _head_body,
        grid=(_B,),
        in_specs=[
            pl.BlockSpec((1, _VP, _H), lambda b: (b, 0, 0)),
            mspec, mspec, mspec,
            pl.BlockSpec((1, _H), lambda b: (0, 0)),
            pl.BlockSpec((_H, _H), lambda b: (0, 0)),
            pl.BlockSpec((1, _H), lambda b: (0, 0)),
            pl.BlockSpec((1, _H), lambda b: (0, 0)),
            pl.BlockSpec((1, 1), lambda b: (0, 0)),
        ],
        out_specs=pl.BlockSpec((1, _VP, 1), lambda b: (b, 0, 0)),
        out_shape=jax.ShapeDtypeStruct((_B, _VP, 1), jnp.float32),
    )(vs_p, m1, m2, cntp, d_r, w1_t, b12, w22, b22)


def _slot_major(ix):
    a = ix.reshape(_NW, 2, 128, _S).transpose(0, 3, 1, 2).reshape(_NW, 6, 128)
    a = jnp.pad(a, ((0, 0), (0, 2), (0, 0)))
    return a.reshape(_NW * 8, 128)


def kernel(clause_indices, variable_states, Wvc, bvc, Wce, bce, Wcv, bcv,
           W1, b1, W2, b2):
    idx = clause_indices
    sidx = _slot_major(idx)
    boff = (jnp.arange(_B, dtype=jnp.int32) * _VP)[:, None, None]
    fidx = _slot_major(idx + boff)
    z2d = jnp.zeros((64, _H), jnp.float32)
    e0 = jnp.zeros((128, _H), jnp.float32).at[:, 0].set(1.0)
    vs_p = jnp.pad(variable_states, ((0, 0), (0, _VP - _V), (0, 0)))

    a_m = (Wvc.T @ Wce.T @ Wcv.T) * (1.0 / _S)
    d_r = (((bvc @ Wce.T) + bce) @ Wcv.T + bcv).reshape(1, _H)
    w1_t = W1.T
    b12 = b1.reshape(1, _H)
    w22 = W2.reshape(1, _H)
    b22 = b2.reshape(1, 1)

    u0 = _u0_call(vs_p, a_m)
    u0f = u0.reshape(_B * _VP, _H)
    msum1, cntp = _sc_fused_c(u0f, fidx, sidx, z2d, e0)
    msum1 = msum1.reshape(_NC, _VP, _H)
    cntp = cntp.reshape(_NC, _VP, _H)

    u1 = _msgu_call(u0, msum1, cntp, a_m, d_r)
    msum2 = _sc_fused_nc(u1.reshape(_B * _VP, _H), fidx, sidx, z2d)
    msum2 = msum2.reshape(_NC, _VP, _H)

    probs = _head_call(vs_p, msum1, msum2, cntp, d_r, w1_t, b12, w22, b22)
    return probs[:, :_V, 0]


# speedup vs baseline: 1.4218x; 1.0649x over previous
import functools

import jax
import jax.numpy as jnp
from jax import lax
from jax.experimental import pallas as pl
from jax.experimental.pallas import tpu as pltpu
from jax.experimental.pallas import tpu_sc as plsc

_B, _C, _S = 4, 2048, 3
_V, _H = 1000, 128
_VP = 1024
_NC, _NS = 2, 16
_NW = _NC * _NS
_NCL = _B * _C
_CPW = _NCL // _NW
_DUMMY = _V
_CW = _H

_SC_MESH = plsc.VectorSubcoreMesh(core_axis_name="c", subcore_axis_name="s")


def _dedup_dests(iv, dv):
    for hh in range(2):
        for k in range(8):
            sl = pl.ds(k * 16, 16)
            i0 = iv[0 + hh, sl]
            i1 = iv[2 + hh, sl]
            i2 = iv[4 + hh, sl]
            dv[hh, sl] = jnp.where(i1 == i0, _DUMMY, i1)
            dv[2 + hh, sl] = jnp.where((i2 == i0) | (i2 == i1), _DUMMY, i2)


def _slot_sum(b0, b1, b2):
    def body(r, carry):
        for k in range(8):
            sl = pl.ds(k * 16, 16)
            b0[r, sl] = b0[r, sl] + b1[r, sl] + b2[r, sl]
        return carry
    lax.fori_loop(0, 128, body, 0)


def _sc_fused_c_body(u_hbm, fidx_hbm, sidx_hbm, z2d_hbm, e0_hbm,
                     msum_hbm, cnt_hbm,
                     idx_v, siv, dv, b0, b1, b2, b3, e0_v,
                     shared_m, shared_c, semi, semg0, semg1):
    cid = lax.axis_index("c")
    sid = lax.axis_index("s")
    chunk = cid * _NS + sid
    pltpu.sync_copy(fidx_hbm.at[pl.ds(chunk * 8, 8)], idx_v)
    pltpu.sync_copy(sidx_hbm.at[pl.ds(chunk * 8, 8)], siv)
    zi0 = pltpu.async_copy(z2d_hbm, shared_m.at[pl.ds(sid * 64, 64)], semi)
    zi1 = pltpu.async_copy(z2d_hbm, shared_c.at[pl.ds(sid * 64, 64)], semi)
    zi2 = pltpu.async_copy(e0_hbm, e0_v, semi)
    g0 = [pltpu.async_copy(u_hbm.at[idx_v.at[s * 2]], b, semg0)
          for s, b in ((0, b0), (1, b1), (2, b2))]
    _dedup_dests(siv, dv)
    for g in g0:
        g.wait()
    _slot_sum(b0, b1, b2)
    g1 = [pltpu.async_copy(u_hbm.at[idx_v.at[s * 2 + 1]], b, semg1)
          for s, b in ((0, b3), (1, b1), (2, b2))]
    zi0.wait()
    zi1.wait()
    zi2.wait()
    plsc.subcore_barrier()
    adds = [
        pltpu.async_copy(b0, shared_m.at[siv.at[0]], semi, add=True),
        pltpu.async_copy(b0, shared_m.at[dv.at[0]], semi, add=True),
        pltpu.async_copy(b0, shared_m.at[dv.at[2]], semi, add=True),
        pltpu.async_copy(e0_v, shared_c.at[siv.at[0]], semi, add=True),
        pltpu.async_copy(e0_v, shared_c.at[dv.at[0]], semi, add=True),
        pltpu.async_copy(e0_v, shared_c.at[dv.at[2]], semi, add=True),
    ]
    for g in g1:
        g.wait()
    _slot_sum(b3, b1, b2)
    adds += [
        pltpu.async_copy(b3, shared_m.at[siv.at[1]], semi, add=True),
        pltpu.async_copy(b3, shared_m.at[dv.at[1]], semi, add=True),
        pltpu.async_copy(b3, shared_m.at[dv.at[3]], semi, add=True),
        pltpu.async_copy(e0_v, shared_c.at[siv.at[1]], semi, add=True),
        pltpu.async_copy(e0_v, shared_c.at[dv.at[1]], semi, add=True),
        pltpu.async_copy(e0_v, shared_c.at[dv.at[3]], semi, add=True),
    ]
    for a in adds:
        a.wait()
    plsc.subcore_barrier()
    pltpu.sync_copy(shared_m.at[pl.ds(sid * 64, 64)],
                    msum_hbm.at[pl.ds(cid * _VP + sid * 64, 64)])
    pltpu.sync_copy(shared_c.at[pl.ds(sid * 64, 64)],
                    cnt_hbm.at[pl.ds(cid * _VP + sid * 64, 64)])


_sc_fused_c = functools.partial(
    pl.kernel,
    mesh=_SC_MESH,
    out_type=[
        jax.ShapeDtypeStruct((_NC * _VP, _H), jnp.float32),
        jax.ShapeDtypeStruct((_NC * _VP, _H), jnp.float32),
    ],
    scratch_types=[
        pltpu.VMEM((8, 128), jnp.int32),
        pltpu.VMEM((8, 128), jnp.int32),
        pltpu.VMEM((4, 128), jnp.int32),
        pltpu.VMEM((128, _H), jnp.float32),
        pltpu.VMEM((128, _H), jnp.float32),
        pltpu.VMEM((128, _H), jnp.float32),
        pltpu.VMEM((128, _H), jnp.float32),
        pltpu.VMEM((128, _H), jnp.float32),
        pltpu.VMEM_SHARED((_VP, _H), jnp.float32),
        pltpu.VMEM_SHARED((_VP, _H), jnp.float32),
        pltpu.SemaphoreType.DMA,
        pltpu.SemaphoreType.DMA,
        pltpu.SemaphoreType.DMA,
    ],
)(_sc_fused_c_body)


def _sc_fused_nc_body(u_hbm, fidx_hbm, sidx_hbm, z2d_hbm, msum_hbm,
                      idx_v, siv, dv, b0, b1, b2, b3,
                      shared_m, semi, semg0, semg1):
    cid = lax.axis_index("c")
    sid = lax.axis_index("s")
    chunk = cid * _NS + sid
    pltpu.sync_copy(fidx_hbm.at[pl.ds(chunk * 8, 8)], idx_v)
    pltpu.sync_copy(sidx_hbm.at[pl.ds(chunk * 8, 8)], siv)
    zi0 = pltpu.async_copy(z2d_hbm, shared_m.at[pl.ds(sid * 64, 64)], semi)
    g0 = [pltpu.async_copy(u_hbm.at[idx_v.at[s * 2]], b, semg0)
          for s, b in ((0, b0), (1, b1), (2, b2))]
    _dedup_dests(siv, dv)
    for g in g0:
        g.wait()
    _slot_sum(b0, b1, b2)
    g1 = [pltpu.async_copy(u_hbm.at[idx_v.at[s * 2 + 1]], b, semg1)
          for s, b in ((0, b3), (1, b1), (2, b2))]
    zi0.wait()
    plsc.subcore_barrier()
    adds = [
        pltpu.async_copy(b0, shared_m.at[siv.at[0]], semi, add=True),
        pltpu.async_copy(b0, shared_m.at[dv.at[0]], semi, add=True),
        pltpu.async_copy(b0, shared_m.at[dv.at[2]], semi, add=True),
    ]
    for g in g1:
        g.wait()
    _slot_sum(b3, b1, b2)
    adds += [
        pltpu.async_copy(b3, shared_m.at[siv.at[1]], semi, add=True),
        pltpu.async_copy(b3, shared_m.at[dv.at[1]], semi, add=True),
        pltpu.async_copy(b3, shared_m.at[dv.at[3]], semi, add=True),
    ]
    for a in adds:
        a.wait()
    plsc.subcore_barrier()
    pltpu.sync_copy(shared_m.at[pl.ds(sid * 64, 64)],
                    msum_hbm.at[pl.ds(cid * _VP + sid * 64, 64)])


_sc_fused_nc = functools.partial(
    pl.kernel,
    mesh=_SC_MESH,
    out_type=jax.ShapeDtypeStruct((_NC * _VP, _H), jnp.float32),
    scratch_types=[
        pltpu.VMEM((8, 128), jnp.int32),
        pltpu.VMEM((8, 128), jnp.int32),
        pltpu.VMEM((4, 128), jnp.int32),
        pltpu.VMEM((128, _H), jnp.float32),
        pltpu.VMEM((128, _H), jnp.float32),
        pltpu.VMEM((128, _H), jnp.float32),
        pltpu.VMEM((128, _H), jnp.float32),
        pltpu.VMEM_SHARED((_VP, _H), jnp.float32),
        pltpu.SemaphoreType.DMA,
        pltpu.SemaphoreType.DMA,
        pltpu.SemaphoreType.DMA,
    ],
)(_sc_fused_nc_body)


def _u0_body(vs_ref, a_ref, out_ref):
    out_ref[0, pl.ds(0, _V), :] = jnp.dot(vs_ref[0], a_ref[...],
                                          preferred_element_type=jnp.float32)
    out_ref[0, pl.ds(_V, _VP - _V), :] = jnp.zeros((_VP - _V, _H), jnp.float32)


def _u0_call(vs, a_m):
    return pl.pallas_call(
        _u0_body,
        grid=(_B,),
        in_specs=[
            pl.BlockSpec((1, _V, _H), lambda b: (b, 0, 0)),
            pl.BlockSpec((_H, _H), lambda b: (0, 0)),
        ],
        out_specs=pl.BlockSpec((1, _VP, _H), lambda b: (b, 0, 0)),
        out_shape=jax.ShapeDtypeStruct((_B, _VP, _H), jnp.float32),
    )(vs, a_m)


def _msgu_body(u0_ref, msum_ref, cntp_ref, a_ref, d_ref, out_ref):
    cs = cntp_ref[0][:, 0:1] + cntp_ref[1][:, 0:1]
    has = (cs > 0).astype(jnp.float32)
    scale = has / jnp.maximum(cs, 1.0)
    msg = (msum_ref[0] + msum_ref[1]) * scale + has * d_ref[...]
    msga = jnp.dot(msg, a_ref[...], preferred_element_type=jnp.float32)
    out_ref[...] = u0_ref[...] + msga[None, :, :]


def _msgu_call(u0, msum, cntp, a_m, d_r):
    return pl.pallas_call(
        _msgu_body,
        out_shape=jax.ShapeDtypeStruct((_B, _VP, _H), jnp.float32),
    )(u0, msum, cntp, a_m, d_r)


def _head_body(vs_ref, m1_ref, m2_ref, cntp_ref, d_ref, w1_ref, b1_ref,
               w2_ref, b2_ref, out_ref):
    cs = cntp_ref[0, 0:_V, 0:1] + cntp_ref[1, 0:_V, 0:1]
    has = (cs > 0).astype(jnp.float32)
    scale = has / jnp.maximum(cs, 1.0)
    m = (m1_ref[0, 0:_V, :] + m1_ref[1, 0:_V, :]
         + m2_ref[0, 0:_V, :] + m2_ref[1, 0:_V, :])
    vsn = vs_ref[0] + m * scale + has * (2.0 * d_ref[...])
    hh = jnp.maximum(
        jnp.dot(vsn, w1_ref[...], preferred_element_type=jnp.float32) + b1_ref[...],
        0.0)
    logit = jnp.sum(hh * w2_ref[...], axis=1, keepdims=True) + b2_ref[...]
    out_ref[0] = jax.nn.sigmoid(logit)


def _head_call(vs, m1, m2, cntp, d_r, w1_t, b12, w22, b22):
    mspec = pl.BlockSpec((2, _VP, _H), lambda b: (0, 0, 0))
    return pl.pallas_call(
        _head_body,
        grid=(_B,),
        in_specs=[
            pl.BlockSpec((1, _V, _H), lambda b: (b, 0, 0)),
            mspec, mspec,
            pl.BlockSpec((2, _VP, _CW), lambda b: (0, 0, 0)),
            pl.BlockSpec((1, _H), lambda b: (0, 0)),
            pl.BlockSpec((_H, _H), lambda b: (0, 0)),
            pl.BlockSpec((1, _H), lambda b: (0, 0)),
            pl.BlockSpec((1, _H), lambda b: (0, 0)),
            pl.BlockSpec((1, 1), lambda b: (0, 0)),
        ],
        out_specs=pl.BlockSpec((1, _V, 1), lambda b: (b, 0, 0)),
        out_shape=jax.ShapeDtypeStruct((_B, _V, 1), jnp.float32),
    )(vs, m1, m2, cntp, d_r, w1_t, b12, w22, b22)


def _slot_major(ix):
    a = ix.reshape(_NW, 2, 128, _S).transpose(0, 3, 1, 2).reshape(_NW, 6, 128)
    a = jnp.pad(a, ((0, 0), (0, 2), (0, 0)))
    return a.reshape(_NW * 8, 128)


def kernel(clause_indices, variable_states, Wvc, bvc, Wce, bce, Wcv, bcv,
           W1, b1, W2, b2):
    idx = clause_indices
    sidx = _slot_major(idx)
    boff = (jnp.arange(_B, dtype=jnp.int32) * _VP)[:, None, None]
    fidx = _slot_major(idx + boff)
    z2d = jnp.zeros((64, _H), jnp.float32)
    e0 = jnp.zeros((128, _CW), jnp.float32).at[:, 0].set(1.0)

    a_m = (Wvc.T @ Wce.T @ Wcv.T) * (1.0 / _S)
    d_r = (((bvc @ Wce.T) + bce) @ Wcv.T + bcv).reshape(1, _H)
    w1_t = W1.T
    b12 = b1.reshape(1, _H)
    w22 = W2.reshape(1, _H)
    b22 = b2.reshape(1, 1)

    u0 = _u0_call(variable_states, a_m)
    u0f = u0.reshape(_B * _VP, _H)
    msum1, cntp = _sc_fused_c(u0f, fidx, sidx, z2d, e0)
    msum1 = msum1.reshape(_NC, _VP, _H)
    cntp = cntp.reshape(_NC, _VP, _CW)

    u1 = _msgu_call(u0, msum1, cntp, a_m, d_r)
    msum2 = _sc_fused_nc(u1.reshape(_B * _VP, _H), fidx, sidx, z2d)
    msum2 = msum2.reshape(_NC, _VP, _H)

    probs = _head_call(variable_states, msum1, msum2, cntp, d_r,
                       w1_t, b12, w22, b22)
    return probs[:, :, 0]


# speedup vs baseline: 1.5834x; 1.1137x over previous
import functools

import jax
import jax.numpy as jnp
from jax import lax
from jax.experimental import pallas as pl
from jax.experimental.pallas import tpu as pltpu
from jax.experimental.pallas import tpu_sc as plsc

_B, _C, _S = 4, 2048, 3
_V, _H = 1000, 128
_VP = 1024
_NC, _NS = 2, 16
_NW = _NC * _NS
_NCL = _B * _C
_CPW = _NCL // _NW
_DUMMY = _V
_CW = _H

_SC_MESH = plsc.VectorSubcoreMesh(core_axis_name="c", subcore_axis_name="s")


def _dedup_dests(iv, dv):
    for hh in range(2):
        for k in range(8):
            sl = pl.ds(k * 16, 16)
            i0 = iv[0 + hh, sl]
            i1 = iv[2 + hh, sl]
            i2 = iv[4 + hh, sl]
            dv[hh, sl] = jnp.where(i1 == i0, _DUMMY, i1)
            dv[2 + hh, sl] = jnp.where((i2 == i0) | (i2 == i1), _DUMMY, i2)


def _slot_sum(b0, b1, b2):
    def body(r, carry):
        for k in range(8):
            sl = pl.ds(k * 16, 16)
            b0[r, sl] = b0[r, sl] + b1[r, sl] + b2[r, sl]
        return carry
    lax.fori_loop(0, 128, body, 0)


def _sc_fused_c_body(u_hbm, fidx_hbm, sidx_hbm, z2d_hbm, e0_hbm,
                     msum_hbm, cnt_hbm,
                     idx_v, siv, dv, b0, b1, b2, b3, e0_v,
                     shared_m, shared_c, semi, semg0, semg1):
    cid = lax.axis_index("c")
    sid = lax.axis_index("s")
    chunk = cid * _NS + sid
    pltpu.sync_copy(fidx_hbm.at[pl.ds(chunk * 8, 8)], idx_v)
    pltpu.sync_copy(sidx_hbm.at[pl.ds(chunk * 8, 8)], siv)
    zi0 = pltpu.async_copy(z2d_hbm, shared_m.at[pl.ds(sid * 64, 64)], semi)
    zi1 = pltpu.async_copy(z2d_hbm, shared_c.at[pl.ds(sid * 64, 64)], semi)
    zi2 = pltpu.async_copy(e0_hbm, e0_v, semi)
    g0 = [pltpu.async_copy(u_hbm.at[idx_v.at[s * 2]], b, semg0)
          for s, b in ((0, b0), (1, b1), (2, b2))]
    _dedup_dests(siv, dv)
    for g in g0:
        g.wait()
    _slot_sum(b0, b1, b2)
    g1 = [pltpu.async_copy(u_hbm.at[idx_v.at[s * 2 + 1]], b, semg1)
          for s, b in ((0, b3), (1, b1), (2, b2))]
    zi0.wait()
    zi1.wait()
    zi2.wait()
    plsc.subcore_barrier()
    adds = [
        pltpu.async_copy(b0, shared_m.at[siv.at[0]], semi, add=True),
        pltpu.async_copy(b0, shared_m.at[dv.at[0]], semi, add=True),
        pltpu.async_copy(b0, shared_m.at[dv.at[2]], semi, add=True),
        pltpu.async_copy(e0_v, shared_c.at[siv.at[0]], semi, add=True),
        pltpu.async_copy(e0_v, shared_c.at[dv.at[0]], semi, add=True),
        pltpu.async_copy(e0_v, shared_c.at[dv.at[2]], semi, add=True),
    ]
    for g in g1:
        g.wait()
    _slot_sum(b3, b1, b2)
    adds += [
        pltpu.async_copy(b3, shared_m.at[siv.at[1]], semi, add=True),
        pltpu.async_copy(b3, shared_m.at[dv.at[1]], semi, add=True),
        pltpu.async_copy(b3, shared_m.at[dv.at[3]], semi, add=True),
        pltpu.async_copy(e0_v, shared_c.at[siv.at[1]], semi, add=True),
        pltpu.async_copy(e0_v, shared_c.at[dv.at[1]], semi, add=True),
        pltpu.async_copy(e0_v, shared_c.at[dv.at[3]], semi, add=True),
    ]
    for a in adds:
        a.wait()
    plsc.subcore_barrier()
    pltpu.sync_copy(shared_m.at[pl.ds(sid * 64, 64)],
                    msum_hbm.at[pl.ds(cid * _VP + sid * 64, 64)])
    pltpu.sync_copy(shared_c.at[pl.ds(sid * 64, 64)],
                    cnt_hbm.at[pl.ds(cid * _VP + sid * 64, 64)])


_sc_fused_c = functools.partial(
    pl.kernel,
    mesh=_SC_MESH,
    out_type=[
        jax.ShapeDtypeStruct((_NC * _VP, _H), jnp.float32),
        jax.ShapeDtypeStruct((_NC * _VP, _H), jnp.float32),
    ],
    scratch_types=[
        pltpu.VMEM((8, 128), jnp.int32),
        pltpu.VMEM((8, 128), jnp.int32),
        pltpu.VMEM((4, 128), jnp.int32),
        pltpu.VMEM((128, _H), jnp.float32),
        pltpu.VMEM((128, _H), jnp.float32),
        pltpu.VMEM((128, _H), jnp.float32),
        pltpu.VMEM((128, _H), jnp.float32),
        pltpu.VMEM((128, _H), jnp.float32),
        pltpu.VMEM_SHARED((_VP, _H), jnp.float32),
        pltpu.VMEM_SHARED((_VP, _H), jnp.float32),
        pltpu.SemaphoreType.DMA,
        pltpu.SemaphoreType.DMA,
        pltpu.SemaphoreType.DMA,
    ],
)(_sc_fused_c_body)


def _sc_fused_nc_body(u_hbm, fidx_hbm, sidx_hbm, z2d_hbm, msum_hbm,
                      idx_v, siv, dv, b0, b1, b2, b3,
                      shared_m, semi, semg0, semg1):
    cid = lax.axis_index("c")
    sid = lax.axis_index("s")
    chunk = cid * _NS + sid
    pltpu.sync_copy(fidx_hbm.at[pl.ds(chunk * 8, 8)], idx_v)
    pltpu.sync_copy(sidx_hbm.at[pl.ds(chunk * 8, 8)], siv)
    zi0 = pltpu.async_copy(z2d_hbm, shared_m.at[pl.ds(sid * 64, 64)], semi)
    g0 = [pltpu.async_copy(u_hbm.at[idx_v.at[s * 2]], b, semg0)
          for s, b in ((0, b0), (1, b1), (2, b2))]
    _dedup_dests(siv, dv)
    for g in g0:
        g.wait()
    _slot_sum(b0, b1, b2)
    g1 = [pltpu.async_copy(u_hbm.at[idx_v.at[s * 2 + 1]], b, semg1)
          for s, b in ((0, b3), (1, b1), (2, b2))]
    zi0.wait()
    plsc.subcore_barrier()
    adds = [
        pltpu.async_copy(b0, shared_m.at[siv.at[0]], semi, add=True),
        pltpu.async_copy(b0, shared_m.at[dv.at[0]], semi, add=True),
        pltpu.async_copy(b0, shared_m.at[dv.at[2]], semi, add=True),
    ]
    for g in g1:
        g.wait()
    _slot_sum(b3, b1, b2)
    adds += [
        pltpu.async_copy(b3, shared_m.at[siv.at[1]], semi, add=True),
        pltpu.async_copy(b3, shared_m.at[dv.at[1]], semi, add=True),
        pltpu.async_copy(b3, shared_m.at[dv.at[3]], semi, add=True),
    ]
    for a in adds:
        a.wait()
    plsc.subcore_barrier()
    pltpu.sync_copy(shared_m.at[pl.ds(sid * 64, 64)],
                    msum_hbm.at[pl.ds(cid * _VP + sid * 64, 64)])


_sc_fused_nc = functools.partial(
    pl.kernel,
    mesh=_SC_MESH,
    out_type=jax.ShapeDtypeStruct((_NC * _VP, _H), jnp.float32),
    scratch_types=[
        pltpu.VMEM((8, 128), jnp.int32),
        pltpu.VMEM((8, 128), jnp.int32),
        pltpu.VMEM((4, 128), jnp.int32),
        pltpu.VMEM((128, _H), jnp.float32),
        pltpu.VMEM((128, _H), jnp.float32),
        pltpu.VMEM((128, _H), jnp.float32),
        pltpu.VMEM((128, _H), jnp.float32),
        pltpu.VMEM_SHARED((_VP, _H), jnp.float32),
        pltpu.SemaphoreType.DMA,
        pltpu.SemaphoreType.DMA,
        pltpu.SemaphoreType.DMA,
    ],
)(_sc_fused_nc_body)


def _fold_x(wvc_ref, wce_ref, wcv_ref):
    x = jnp.dot(wcv_ref[...], wce_ref[...], preferred_element_type=jnp.float32)
    return jnp.dot(x, wvc_ref[...], preferred_element_type=jnp.float32)


def _fold_d(bvc_ref, bce_ref, bcv_ref, wce_ref, wcv_ref):
    t = lax.dot_general(bvc_ref[...], wce_ref[...], (((1,), (1,)), ((), ())),
                        preferred_element_type=jnp.float32) + bce_ref[...]
    return lax.dot_general(t, wcv_ref[...], (((1,), (1,)), ((), ())),
                           preferred_element_type=jnp.float32) + bcv_ref[...]


def _u0_body(vs_ref, wvc_ref, wce_ref, wcv_ref, out_ref, z2d_ref, e0_ref):
    x = _fold_x(wvc_ref, wce_ref, wcv_ref)
    u = lax.dot_general(vs_ref[0], x, (((1,), (1,)), ((), ())),
                        preferred_element_type=jnp.float32) * (1.0 / _S)
    out_ref[0, pl.ds(0, _V), :] = u
    out_ref[0, pl.ds(_V, _VP - _V), :] = jnp.zeros((_VP - _V, _H), jnp.float32)

    @pl.when(pl.program_id(0) == 0)
    def _consts():
        z2d_ref[...] = jnp.zeros((64, _H), jnp.float32)
        lane = jax.lax.broadcasted_iota(jnp.int32, (128, _H), 1)
        e0_ref[...] = (lane == 0).astype(jnp.float32)


def _u0_call(vs, wvc, wce, wcv):
    wspec = pl.BlockSpec((_H, _H), lambda b: (0, 0))
    return pl.pallas_call(
        _u0_body,
        grid=(_B,),
        in_specs=[pl.BlockSpec((1, _V, _H), lambda b: (b, 0, 0)),
                  wspec, wspec, wspec],
        out_specs=[
            pl.BlockSpec((1, _VP, _H), lambda b: (b, 0, 0)),
            pl.BlockSpec((64, _H), lambda b: (0, 0)),
            pl.BlockSpec((128, _H), lambda b: (0, 0)),
        ],
        out_shape=[
            jax.ShapeDtypeStruct((_B, _VP, _H), jnp.float32),
            jax.ShapeDtypeStruct((64, _H), jnp.float32),
            jax.ShapeDtypeStruct((128, _H), jnp.float32),
        ],
    )(vs, wvc, wce, wcv)


def _msgu_body(u0_ref, msum_ref, cntp_ref, wvc_ref, wce_ref, wcv_ref,
               bvc_ref, bce_ref, bcv_ref, out_ref):
    cs = cntp_ref[0][:, 0:1] + cntp_ref[1][:, 0:1]
    has = (cs > 0).astype(jnp.float32)
    scale = has / jnp.maximum(cs, 1.0)
    d = _fold_d(bvc_ref, bce_ref, bcv_ref, wce_ref, wcv_ref)
    msg = (msum_ref[0] + msum_ref[1]) * scale + has * d
    x = _fold_x(wvc_ref, wce_ref, wcv_ref)
    msga = lax.dot_general(msg, x, (((1,), (1,)), ((), ())),
                           preferred_element_type=jnp.float32) * (1.0 / _S)
    out_ref[...] = u0_ref[...] + msga[None, :, :]


def _msgu_call(u0, msum, cntp, wvc, wce, wcv, bvc2, bce2, bcv2):
    return pl.pallas_call(
        _msgu_body,
        out_shape=jax.ShapeDtypeStruct((_B, _VP, _H), jnp.float32),
    )(u0, msum, cntp, wvc, wce, wcv, bvc2, bce2, bcv2)


def _head_body(vs_ref, m1_ref, m2_ref, cntp_ref, wce_ref, wcv_ref,
               bvc_ref, bce_ref, bcv_ref, w1_ref, b1_ref, w2_ref, b2_ref,
               out_ref):
    cs = cntp_ref[0, 0:_V, 0:1] + cntp_ref[1, 0:_V, 0:1]
    has = (cs > 0).astype(jnp.float32)
    scale = has / jnp.maximum(cs, 1.0)
    d = _fold_d(bvc_ref, bce_ref, bcv_ref, wce_ref, wcv_ref)
    m = (m1_ref[0, 0:_V, :] + m1_ref[1, 0:_V, :]
         + m2_ref[0, 0:_V, :] + m2_ref[1, 0:_V, :])
    upd = m * scale + has * (2.0 * d)
    vsn = vs_ref[...] + upd[None, :, :]
    hh = jnp.maximum(
        lax.dot_general(vsn, w1_ref[...], (((2,), (1,)), ((), ())),
                        preferred_element_type=jnp.float32) + b1_ref[...],
        0.0)
    rows = [lax.dot_general(w2_ref[...], hh[b], (((1,), (1,)), ((), ())),
                            preferred_element_type=jnp.float32)
            for b in range(_B)]
    logit = jnp.concatenate(rows, axis=0) + b2_ref[...]
    out_ref[...] = jax.nn.sigmoid(logit)


def _head_call(vs, m1, m2, cntp, wce, wcv, bvc2, bce2, bcv2, w1, b12, w22,
               b22):
    return pl.pallas_call(
        _head_body,
        out_shape=jax.ShapeDtypeStruct((_B, _V), jnp.float32),
    )(vs, m1, m2, cntp, wce, wcv, bvc2, bce2, bcv2, w1, b12, w22, b22)


def _slot_major(ix):
    a = ix.reshape(_NW, 2, 128, _S).transpose(0, 3, 1, 2).reshape(_NW, 6, 128)
    a = jnp.pad(a, ((0, 0), (0, 2), (0, 0)))
    return a.reshape(_NW * 8, 128)


def kernel(clause_indices, variable_states, Wvc, bvc, Wce, bce, Wcv, bcv,
           W1, b1, W2, b2):
    idx = clause_indices
    sidx = _slot_major(idx)
    boff = (jnp.arange(_B, dtype=jnp.int32) * _VP)[:, None, None]
    fidx = _slot_major(idx + boff)

    bvc2, bce2, bcv2, b12 = (x.reshape(1, _H) for x in (bvc, bce, bcv, b1))
    w22 = W2.reshape(1, _H)
    b22 = b2.reshape(1, 1)

    u0, z2d, e0 = _u0_call(variable_states, Wvc, Wce, Wcv)
    u0f = u0.reshape(_B * _VP, _H)
    msum1, cntp = _sc_fused_c(u0f, fidx, sidx, z2d, e0)
    msum1 = msum1.reshape(_NC, _VP, _H)
    cntp = cntp.reshape(_NC, _VP, _CW)

    u1 = _msgu_call(u0, msum1, cntp, Wvc, Wce, Wcv, bvc2, bce2, bcv2)
    msum2 = _sc_fused_nc(u1.reshape(_B * _VP, _H), fidx, sidx, z2d)
    msum2 = msum2.reshape(_NC, _VP, _H)

    return _head_call(variable_states, msum1, msum2, cntp, Wce, Wcv,
                      bvc2, bce2, bcv2, W1, b12, w22, b22)


# speedup vs baseline: 1.6047x; 1.0134x over previous
import functools

import jax
import jax.numpy as jnp
from jax import lax
from jax.experimental import pallas as pl
from jax.experimental.pallas import tpu as pltpu
from jax.experimental.pallas import tpu_sc as plsc

_B, _C, _S = 4, 2048, 3
_V, _H = 1000, 128
_VP = 1024
_NC, _NS = 2, 16
_NW = _NC * _NS
_NCL = _B * _C
_CPW = _NCL // _NW
_DUMMY = _V
_CW = _H

_SC_MESH = plsc.VectorSubcoreMesh(core_axis_name="c", subcore_axis_name="s")


def _dedup_dests(iv, dv):
    for hh in range(2):
        for k in range(8):
            sl = pl.ds(k * 16, 16)
            i0 = iv[0 + hh, sl]
            i1 = iv[2 + hh, sl]
            i2 = iv[4 + hh, sl]
            dv[hh, sl] = jnp.where(i1 == i0, _DUMMY, i1)
            dv[2 + hh, sl] = jnp.where((i2 == i0) | (i2 == i1), _DUMMY, i2)


def _slot_sum(b0, b1, b2):
    def body(r, carry):
        for k in range(8):
            sl = pl.ds(k * 16, 16)
            b0[r, sl] = b0[r, sl] + b1[r, sl] + b2[r, sl]
        return carry
    lax.fori_loop(0, 128, body, 0)


def _sc_fused_c_body(u_hbm, fidx_hbm, sidx_hbm, z2d_hbm, e0_hbm,
                     msum_hbm, cnt_hbm,
                     idx_v, siv, dv, b0, b1, b2, b3, e0_v,
                     shared_m, shared_c, semi, semg0, semg1):
    cid = lax.axis_index("c")
    sid = lax.axis_index("s")
    chunk = cid * _NS + sid
    li0 = pltpu.async_copy(fidx_hbm.at[pl.ds(chunk * 8, 8)], idx_v, semg0)
    li1 = pltpu.async_copy(sidx_hbm.at[pl.ds(chunk * 8, 8)], siv, semg1)
    zi0 = pltpu.async_copy(z2d_hbm, shared_m.at[pl.ds(sid * 64, 64)], semi)
    zi1 = pltpu.async_copy(z2d_hbm, shared_c.at[pl.ds(sid * 64, 64)], semi)
    zi2 = pltpu.async_copy(e0_hbm, e0_v, semi)
    li0.wait()
    g0 = [pltpu.async_copy(u_hbm.at[idx_v.at[s * 2]], b, semg0)
          for s, b in ((0, b0), (1, b1), (2, b2))]
    li1.wait()
    _dedup_dests(siv, dv)
    for g in g0:
        g.wait()
    _slot_sum(b0, b1, b2)
    g1 = [pltpu.async_copy(u_hbm.at[idx_v.at[s * 2 + 1]], b, semg1)
          for s, b in ((0, b3), (1, b1), (2, b2))]
    zi0.wait()
    zi1.wait()
    zi2.wait()
    plsc.subcore_barrier()
    adds = [
        pltpu.async_copy(b0, shared_m.at[siv.at[0]], semi, add=True),
        pltpu.async_copy(b0, shared_m.at[dv.at[0]], semi, add=True),
        pltpu.async_copy(b0, shared_m.at[dv.at[2]], semi, add=True),
        pltpu.async_copy(e0_v, shared_c.at[siv.at[0]], semi, add=True),
        pltpu.async_copy(e0_v, shared_c.at[dv.at[0]], semi, add=True),
        pltpu.async_copy(e0_v, shared_c.at[dv.at[2]], semi, add=True),
    ]
    for g in g1:
        g.wait()
    _slot_sum(b3, b1, b2)
    adds += [
        pltpu.async_copy(b3, shared_m.at[siv.at[1]], semi, add=True),
        pltpu.async_copy(b3, shared_m.at[dv.at[1]], semi, add=True),
        pltpu.async_copy(b3, shared_m.at[dv.at[3]], semi, add=True),
        pltpu.async_copy(e0_v, shared_c.at[siv.at[1]], semi, add=True),
        pltpu.async_copy(e0_v, shared_c.at[dv.at[1]], semi, add=True),
        pltpu.async_copy(e0_v, shared_c.at[dv.at[3]], semi, add=True),
    ]
    for a in adds:
        a.wait()
    plsc.subcore_barrier()
    co0 = pltpu.async_copy(shared_m.at[pl.ds(sid * 64, 64)],
                           msum_hbm.at[pl.ds(cid * _VP + sid * 64, 64)], semg0)
    co1 = pltpu.async_copy(shared_c.at[pl.ds(sid * 64, 64)],
                           cnt_hbm.at[pl.ds(cid * _VP + sid * 64, 64)], semg1)
    co0.wait()
    co1.wait()


_sc_fused_c = functools.partial(
    pl.kernel,
    mesh=_SC_MESH,
    out_type=[
        jax.ShapeDtypeStruct((_NC * _VP, _H), jnp.float32),
        jax.ShapeDtypeStruct((_NC * _VP, _H), jnp.float32),
    ],
    scratch_types=[
        pltpu.VMEM((8, 128), jnp.int32),
        pltpu.VMEM((8, 128), jnp.int32),
        pltpu.VMEM((4, 128), jnp.int32),
        pltpu.VMEM((128, _H), jnp.float32),
        pltpu.VMEM((128, _H), jnp.float32),
        pltpu.VMEM((128, _H), jnp.float32),
        pltpu.VMEM((128, _H), jnp.float32),
        pltpu.VMEM((128, _H), jnp.float32),
        pltpu.VMEM_SHARED((_VP, _H), jnp.float32),
        pltpu.VMEM_SHARED((_VP, _H), jnp.float32),
        pltpu.SemaphoreType.DMA,
        pltpu.SemaphoreType.DMA,
        pltpu.SemaphoreType.DMA,
    ],
)(_sc_fused_c_body)


def _sc_fused_nc_body(u_hbm, fidx_hbm, sidx_hbm, z2d_hbm, msum_hbm,
                      idx_v, siv, dv, b0, b1, b2, b3, b4, b5,
                      shared_m, semi, semg0, semg1):
    cid = lax.axis_index("c")
    sid = lax.axis_index("s")
    chunk = cid * _NS + sid
    li0 = pltpu.async_copy(fidx_hbm.at[pl.ds(chunk * 8, 8)], idx_v, semg0)
    li1 = pltpu.async_copy(sidx_hbm.at[pl.ds(chunk * 8, 8)], siv, semg1)
    zi0 = pltpu.async_copy(z2d_hbm, shared_m.at[pl.ds(sid * 64, 64)], semi)
    li0.wait()
    g0 = [pltpu.async_copy(u_hbm.at[idx_v.at[s * 2]], b, semg0)
          for s, b in ((0, b0), (1, b1), (2, b2))]
    li1.wait()
    g1 = [pltpu.async_copy(u_hbm.at[idx_v.at[s * 2 + 1]], b, semg1)
          for s, b in ((0, b3), (1, b4), (2, b5))]
    _dedup_dests(siv, dv)
    for g in g0:
        g.wait()
    _slot_sum(b0, b1, b2)
    zi0.wait()
    plsc.subcore_barrier()
    adds = [
        pltpu.async_copy(b0, shared_m.at[siv.at[0]], semi, add=True),
        pltpu.async_copy(b0, shared_m.at[dv.at[0]], semi, add=True),
        pltpu.async_copy(b0, shared_m.at[dv.at[2]], semi, add=True),
    ]
    for g in g1:
        g.wait()
    _slot_sum(b3, b4, b5)
    adds += [
        pltpu.async_copy(b3, shared_m.at[siv.at[1]], semi, add=True),
        pltpu.async_copy(b3, shared_m.at[dv.at[1]], semi, add=True),
        pltpu.async_copy(b3, shared_m.at[dv.at[3]], semi, add=True),
    ]
    for a in adds:
        a.wait()
    plsc.subcore_barrier()
    pltpu.sync_copy(shared_m.at[pl.ds(sid * 64, 64)],
                    msum_hbm.at[pl.ds(cid * _VP + sid * 64, 64)])


_sc_fused_nc = functools.partial(
    pl.kernel,
    mesh=_SC_MESH,
    out_type=jax.ShapeDtypeStruct((_NC * _VP, _H), jnp.float32),
    scratch_types=[
        pltpu.VMEM((8, 128), jnp.int32),
        pltpu.VMEM((8, 128), jnp.int32),
        pltpu.VMEM((4, 128), jnp.int32),
        pltpu.VMEM((128, _H), jnp.float32),
        pltpu.VMEM((128, _H), jnp.float32),
        pltpu.VMEM((128, _H), jnp.float32),
        pltpu.VMEM((128, _H), jnp.float32),
        pltpu.VMEM((128, _H), jnp.float32),
        pltpu.VMEM((128, _H), jnp.float32),
        pltpu.VMEM_SHARED((_VP, _H), jnp.float32),
        pltpu.SemaphoreType.DMA,
        pltpu.SemaphoreType.DMA,
        pltpu.SemaphoreType.DMA,
    ],
)(_sc_fused_nc_body)


def _fold_x(wvc_ref, wce_ref, wcv_ref):
    x = jnp.dot(wcv_ref[...], wce_ref[...], preferred_element_type=jnp.float32)
    return jnp.dot(x, wvc_ref[...], preferred_element_type=jnp.float32)


def _fold_d(bvc_ref, bce_ref, bcv_ref, wce_ref, wcv_ref):
    t = lax.dot_general(bvc_ref[...], wce_ref[...], (((1,), (1,)), ((), ())),
                        preferred_element_type=jnp.float32) + bce_ref[...]
    return lax.dot_general(t, wcv_ref[...], (((1,), (1,)), ((), ())),
                           preferred_element_type=jnp.float32) + bcv_ref[...]


def _u0_body(vs_ref, wvc_ref, wce_ref, wcv_ref, out_ref, z2d_ref, e0_ref):
    x = _fold_x(wvc_ref, wce_ref, wcv_ref)
    u = lax.dot_general(vs_ref[0], x, (((1,), (1,)), ((), ())),
                        preferred_element_type=jnp.float32) * (1.0 / _S)
    out_ref[0, pl.ds(0, _V), :] = u
    out_ref[0, pl.ds(_V, _VP - _V), :] = jnp.zeros((_VP - _V, _H), jnp.float32)

    @pl.when(pl.program_id(0) == 0)
    def _consts():
        z2d_ref[...] = jnp.zeros((64, _H), jnp.float32)
        lane = jax.lax.broadcasted_iota(jnp.int32, (128, _H), 1)
        e0_ref[...] = (lane == 0).astype(jnp.float32)


def _u0_call(vs, wvc, wce, wcv):
    wspec = pl.BlockSpec((_H, _H), lambda b: (0, 0))
    return pl.pallas_call(
        _u0_body,
        grid=(_B,),
        in_specs=[pl.BlockSpec((1, _V, _H), lambda b: (b, 0, 0)),
                  wspec, wspec, wspec],
        out_specs=[
            pl.BlockSpec((1, _VP, _H), lambda b: (b, 0, 0)),
            pl.BlockSpec((64, _H), lambda b: (0, 0)),
            pl.BlockSpec((128, _H), lambda b: (0, 0)),
        ],
        out_shape=[
            jax.ShapeDtypeStruct((_B, _VP, _H), jnp.float32),
            jax.ShapeDtypeStruct((64, _H), jnp.float32),
            jax.ShapeDtypeStruct((128, _H), jnp.float32),
        ],
    )(vs, wvc, wce, wcv)


def _msgu_body(u0_ref, msum_ref, cntp_ref, wvc_ref, wce_ref, wcv_ref,
               bvc_ref, bce_ref, bcv_ref, out_ref):
    cs = cntp_ref[0][:, 0:1] + cntp_ref[1][:, 0:1]
    has = (cs > 0).astype(jnp.float32)
    scale = has / jnp.maximum(cs, 1.0)
    d = _fold_d(bvc_ref, bce_ref, bcv_ref, wce_ref, wcv_ref)
    msg = (msum_ref[0] + msum_ref[1]) * scale + has * d
    x = _fold_x(wvc_ref, wce_ref, wcv_ref)
    msga = lax.dot_general(msg, x, (((1,), (1,)), ((), ())),
                           preferred_element_type=jnp.float32) * (1.0 / _S)
    out_ref[...] = u0_ref[...] + msga[None, :, :]


def _msgu_call(u0, msum, cntp, wvc, wce, wcv, bvc2, bce2, bcv2):
    return pl.pallas_call(
        _msgu_body,
        out_shape=jax.ShapeDtypeStruct((_B, _VP, _H), jnp.float32),
    )(u0, msum, cntp, wvc, wce, wcv, bvc2, bce2, bcv2)


def _head_body(vs_ref, m1_ref, m2_ref, cntp_ref, wce_ref, wcv_ref,
               bvc_ref, bce_ref, bcv_ref, w1_ref, b1_ref, w2_ref, b2_ref,
               out_ref):
    cs = cntp_ref[0, 0:_V, 0:1] + cntp_ref[1, 0:_V, 0:1]
    has = (cs > 0).astype(jnp.float32)
    scale = has / jnp.maximum(cs, 1.0)
    d = _fold_d(bvc_ref, bce_ref, bcv_ref, wce_ref, wcv_ref)
    m = (m1_ref[0, 0:_V, :] + m1_ref[1, 0:_V, :]
         + m2_ref[0, 0:_V, :] + m2_ref[1, 0:_V, :])
    upd = m * scale + has * (2.0 * d)
    vsn = vs_ref[...] + upd[None, :, :]
    hh = jnp.maximum(
        lax.dot_general(vsn, w1_ref[...], (((2,), (1,)), ((), ())),
                        preferred_element_type=jnp.float32) + b1_ref[...],
        0.0)
    rows = [lax.dot_general(w2_ref[...], hh[b], (((1,), (1,)), ((), ())),
                            preferred_element_type=jnp.float32)
            for b in range(_B)]
    logit = jnp.concatenate(rows, axis=0) + b2_ref[...]
    out_ref[...] = jax.nn.sigmoid(logit)


def _head_call(vs, m1, m2, cntp, wce, wcv, bvc2, bce2, bcv2, w1, b12, w22,
               b22):
    return pl.pallas_call(
        _head_body,
        out_shape=jax.ShapeDtypeStruct((_B, _V), jnp.float32),
    )(vs, m1, m2, cntp, wce, wcv, bvc2, bce2, bcv2, w1, b12, w22, b22)


def _slot_major(ix):
    a = ix.reshape(_NW, 2, 128, _S).transpose(0, 3, 1, 2).reshape(_NW, 6, 128)
    a = jnp.pad(a, ((0, 0), (0, 2), (0, 0)))
    return a.reshape(_NW * 8, 128)


def kernel(clause_indices, variable_states, Wvc, bvc, Wce, bce, Wcv, bcv,
           W1, b1, W2, b2):
    idx = clause_indices
    sidx = _slot_major(idx)
    boff = (jnp.arange(_B, dtype=jnp.int32) * _VP)[:, None, None]
    fidx = _slot_major(idx + boff)

    bvc2, bce2, bcv2, b12 = (x.reshape(1, _H) for x in (bvc, bce, bcv, b1))
    w22 = W2.reshape(1, _H)
    b22 = b2.reshape(1, 1)

    u0, z2d, e0 = _u0_call(variable_states, Wvc, Wce, Wcv)
    u0f = u0.reshape(_B * _VP, _H)
    msum1, cntp = _sc_fused_c(u0f, fidx, sidx, z2d, e0)
    msum1 = msum1.reshape(_NC, _VP, _H)
    cntp = cntp.reshape(_NC, _VP, _CW)

    u1 = _msgu_call(u0, msum1, cntp, Wvc, Wce, Wcv, bvc2, bce2, bcv2)
    msum2 = _sc_fused_nc(u1.reshape(_B * _VP, _H), fidx, sidx, z2d)
    msum2 = msum2.reshape(_NC, _VP, _H)

    return _head_call(variable_states, msum1, msum2, cntp, Wce, Wcv,
                      bvc2, bce2, bcv2, W1, b12, w22, b22)
